# baseline (speedup 1.0000x reference)
import math

import jax
import jax.numpy as jnp
import numpy as np
from jax import lax
from jax.experimental import pallas as pl
from jax.experimental.pallas import tpu as pltpu

SEQ = 16
DIM = 32
NQ = 3
NWORD = 32
WIN = 2
KTAP = 2 * WIN + 1
LCNN = 3
LOUT = 3
LN_EPS = 1e-5


def _layernorm(x, g, b):
    mu = jnp.mean(x, axis=-1, keepdims=True)
    var = jnp.mean((x - mu) ** 2, axis=-1, keepdims=True)
    return (x - mu) * lax.rsqrt(var + LN_EPS) * g + b


def _gelu(x):
    return 0.5 * x * (1.0 + lax.erf(x * 0.7071067811865476))


def _band_cat(conv_w):
    w = conv_w.reshape(LCNN, KTAP, KTAP)
    c = jnp.arange(DIM)[:, None]
    d = jnp.arange(DIM)[None, :]
    dj = c - d + WIN
    valid = (dj >= 0) & (dj < KTAP)
    djc = jnp.clip(dj, 0, KTAP - 1)
    band = jnp.where(valid[None, None], w[:, :, djc], 0.0)
    return band.transpose(0, 2, 1, 3).reshape(LCNN, DIM, KTAP * DIM)


def _shift_lanes(x, sh):
    if sh == 0:
        return x
    nl = x.shape[1]
    z = jnp.zeros((x.shape[0], abs(sh)), x.dtype)
    if sh > 0:
        return jnp.concatenate([x[:, sh:], z], axis=1)
    return jnp.concatenate([z, x[:, :nl + sh]], axis=1)


def _dti_block_kernel(phar_ref, mol_ref, prot_ref, packc_ref, wat_ref,
                      matsn_ref, vec_ref, vect_ref, packh_ref, out_ref):
    bb = phar_ref.shape[0]
    n = SEQ * bb
    f32 = jnp.float32

    idx = prot_ref[0]
    iota = lax.broadcasted_iota(jnp.int32, (NWORD, n), 0)
    xs = (idx == iota).astype(f32)
    ba_col = vect_ref[:, 0:1]
    for l in range(LCNN):
        stack = jnp.concatenate(
            [_shift_lanes(xs, (di - WIN) * bb) for di in range(KTAP)],
            axis=0)
        wc = packc_ref[l * DIM:(l + 1) * DIM, :]
        xs = jnp.maximum(
            jnp.dot(wc, stack, preferred_element_type=f32)
            + vect_ref[:, 1 + l:2 + l], 0.0)

    p = phar_ref[...]
    h1 = _gelu(jnp.dot(p, matsn_ref[0:NQ * DIM, :],
                       preferred_element_type=f32) + vec_ref[0:1, :])
    prompt = jnp.dot(h1, matsn_ref[NQ * DIM:NQ * DIM + DIM, :],
                     preferred_element_type=f32) + vec_ref[1:2, :]
    mol = _layernorm(prompt + mol_ref[...], vec_ref[2:3, :], vec_ref[3:4, :])
    molt = jnp.transpose(mol)

    wat = wat_ref[...]
    ht = jnp.maximum(jnp.dot(wat, molt, preferred_element_type=f32)
                     + ba_col, 0.0)
    hst = jnp.maximum(jnp.dot(wat, xs, preferred_element_type=f32)
                      + ba_col, 0.0)
    ht_tiled = jnp.concatenate([ht] * SEQ, axis=1)
    ones_d = jnp.ones((1, DIM), f32)
    sig = jnp.dot(ones_d, ht_tiled * hst, preferred_element_type=f32)
    wts = jnp.tanh(sig)
    wprod = wts * hst
    prott = wprod[:, 0:bb]
    for s in range(1, SEQ):
        prott = prott + wprod[:, s * bb:(s + 1) * bb]
    prott = prott * (1.0 / SEQ)

    D2 = 2 * DIM
    cat = jnp.maximum(
        jnp.dot(packh_ref[0:D2, 0:DIM], molt, preferred_element_type=f32)
        + jnp.dot(packh_ref[0:D2, DIM:D2], prott, preferred_element_type=f32)
        + packh_ref[LOUT * D2:LOUT * D2 + D2, 0:1], 0.0)
    for j in range(1, LOUT):
        wjt = packh_ref[j * D2:(j + 1) * D2, :]
        cat = jnp.maximum(
            jnp.dot(wjt, cat, preferred_element_type=f32)
            + packh_ref[LOUT * D2:LOUT * D2 + D2, j:j + 1],
            0.0)

    ones_2d = jnp.ones((1, D2), f32)
    wint_col = packh_ref[LOUT * D2:LOUT * D2 + D2, LOUT:LOUT + 1]
    out = (jnp.dot(ones_2d, cat * wint_col, preferred_element_type=f32)
           + vec_ref[4:5, 0:1])
    out_ref[...] = out


def _forward_impl(phar_prompt, mol_repr, protein_batch, proj_w1, proj_b1, proj_w2,
             proj_b2, emb, mol_gamma, mol_beta, prot_gamma, prot_beta, conv_w,
             conv_b, wa, ba, wout_w, wout_b, wint_w, wint_b):
    bn = mol_repr.shape[0]
    bb = math.gcd(bn, 2048)
    nblk = bn // bb

    phar2 = phar_prompt.reshape(bn, NQ * DIM)
    prot_flat = protein_batch.reshape(nblk, bb, SEQ).transpose(0, 2, 1) \
                             .reshape(nblk, 1, SEQ * bb)

    band = _band_cat(conv_w)
    emb_ln = _layernorm(emb, prot_gamma, prot_beta)
    t1 = jnp.dot(emb_ln, band[0])
    rows = []
    for l in range(LCNN):
        base = t1 if l == 0 else band[l]
        rows.append(jnp.concatenate(
            [base[:, di * DIM:(di + 1) * DIM].T for di in range(KTAP)],
            axis=1))
    packc = jnp.concatenate(rows, axis=0)
    wat = wa.T

    matsn = jnp.concatenate([proj_w1, proj_w2],
                            axis=0)
    vec = jnp.concatenate([
        proj_b1, proj_b2, mol_gamma, mol_beta,
        jnp.pad(wint_b, ((0, 0), (0, DIM - 1))),
    ], axis=0)
    vect = jnp.concatenate([
        ba.T,
        jnp.broadcast_to(conv_b[0], (DIM, 1)),
        jnp.broadcast_to(conv_b[1], (DIM, 1)),
        jnp.broadcast_to(conv_b[2], (DIM, 1)),
        jnp.zeros((DIM, 1), jnp.float32),
    ], axis=1)

    D2 = 2 * DIM
    headmats = jnp.concatenate(
        [wout_w[j].T for j in range(LOUT)], axis=0)
    bias_cols = jnp.concatenate(
        [wout_b[j].T for j in range(LOUT)] + [wint_w], axis=1)
    packh = jnp.concatenate([
        headmats,
        jnp.pad(bias_cols, ((0, 0), (0, D2 - (LOUT + 1)))),
    ], axis=0)

    out = pl.pallas_call(
        _dti_block_kernel,
        out_shape=jax.ShapeDtypeStruct((1, bn), jnp.float32),
        grid=(nblk,),
        in_specs=[
            pl.BlockSpec((bb, NQ * DIM), lambda b: (b, 0)),
            pl.BlockSpec((bb, DIM), lambda b: (b, 0)),
            pl.BlockSpec((1, 1, SEQ * bb), lambda b: (b, 0, 0)),
            pl.BlockSpec((LCNN * DIM, KTAP * DIM), lambda b: (0, 0)),
            pl.BlockSpec((DIM, DIM), lambda b: (0, 0)),
            pl.BlockSpec(((NQ + 1) * DIM, DIM), lambda b: (0, 0)),
            pl.BlockSpec((5, DIM), lambda b: (0, 0)),
            pl.BlockSpec((DIM, 5), lambda b: (0, 0)),
            pl.BlockSpec((4 * D2, D2), lambda b: (0, 0)),
        ],
        out_specs=pl.BlockSpec((1, bb), lambda b: (0, b)),
        compiler_params=pltpu.CompilerParams(
            dimension_semantics=("parallel",)),
    )(phar2, mol_repr, prot_flat, packc, wat, matsn, vec, vect, packh)
    return out.reshape(bn, 1)


try:
    from jax import shard_map as _shard_map_fn

    def _shard_map(f, mesh, in_specs, out_specs):
        return _shard_map_fn(f, mesh=mesh, in_specs=in_specs,
                             out_specs=out_specs, check_vma=False)
except ImportError:
    from jax.experimental.shard_map import shard_map as _shard_map_fn

    def _shard_map(f, mesh, in_specs, out_specs):
        try:
            return _shard_map_fn(f, mesh=mesh, in_specs=in_specs,
                                 out_specs=out_specs, check_rep=False)
        except TypeError:
            return _shard_map_fn(f, mesh=mesh, in_specs=in_specs,
                                 out_specs=out_specs)


@jax.jit
def _forward(*args):
    bn = args[1].shape[0]
    devs = jax.devices()
    if len(devs) < 2 or bn % 4096 != 0:
        return _forward_impl(*args)
    mesh = jax.sharding.Mesh(np.array(devs[:2]), ("x",))
    P = jax.sharding.PartitionSpec
    specs = (P("x"), P("x"), P("x")) + tuple(P() for _ in range(17))
    f = _shard_map(_forward_impl, mesh, specs, P("x"))
    return f(*args)


def kernel(phar_prompt, mol_repr, protein_batch, proj_w1, proj_b1, proj_w2,
           proj_b2, emb, mol_gamma, mol_beta, prot_gamma, prot_beta, conv_w,
           conv_b, wa, ba, wout_w, wout_b, wint_w, wint_b):
    return _forward(phar_prompt, mol_repr, protein_batch, proj_w1, proj_b1,
                    proj_w2, proj_b2, emb, mol_gamma, mol_beta, prot_gamma,
                    prot_beta, conv_w, conv_b, wa, ba, wout_w, wout_b,
                    wint_w, wint_b)


# speedup vs baseline: 1.4962x; 1.4962x over previous
import math

import jax
import jax.numpy as jnp
import numpy as np
from jax import lax
from jax.experimental import pallas as pl
from jax.experimental.pallas import tpu as pltpu

SEQ = 16
DIM = 32
NQ = 3
NWORD = 32
WIN = 2
KTAP = 2 * WIN + 1
LCNN = 3
LOUT = 3
LN_EPS = 1e-5


def _layernorm(x, g, b):
    mu = jnp.mean(x, axis=-1, keepdims=True)
    var = jnp.mean((x - mu) ** 2, axis=-1, keepdims=True)
    return (x - mu) * lax.rsqrt(var + LN_EPS) * g + b


def _gelu(x):
    return 0.5 * x * (1.0 + lax.erf(x * 0.7071067811865476))


def _band_cat(conv_w):
    w = conv_w.reshape(LCNN, KTAP, KTAP)
    c = jnp.arange(DIM)[:, None]
    d = jnp.arange(DIM)[None, :]
    dj = c - d + WIN
    valid = (dj >= 0) & (dj < KTAP)
    djc = jnp.clip(dj, 0, KTAP - 1)
    band = jnp.where(valid[None, None], w[:, :, djc], 0.0)
    return band.transpose(0, 2, 1, 3).reshape(LCNN, DIM, KTAP * DIM)


def _shift_lanes(x, sh):
    if sh == 0:
        return x
    nl = x.shape[1]
    z = jnp.zeros((x.shape[0], abs(sh)), x.dtype)
    if sh > 0:
        return jnp.concatenate([x[:, sh:], z], axis=1)
    return jnp.concatenate([z, x[:, :nl + sh]], axis=1)


def _dti_block_kernel(phar_ref, mol_ref, prot_ref, packc_ref, wat_ref,
                      matsn_ref, vec_ref, vect_ref, packh_ref, out_ref):
    bb = phar_ref.shape[0]
    n = SEQ * bb
    f32 = jnp.float32

    idx = prot_ref[0]
    iota = lax.broadcasted_iota(jnp.int32, (NWORD, n), 0)
    xs = (idx == iota).astype(f32)
    ba_col = vect_ref[:, 0:1]
    for l in range(LCNN):
        stack = jnp.concatenate(
            [_shift_lanes(xs, (di - WIN) * bb) for di in range(KTAP)],
            axis=0)
        wc = packc_ref[l * DIM:(l + 1) * DIM, :]
        xs = jnp.maximum(
            jnp.dot(wc, stack, preferred_element_type=f32)
            + vect_ref[:, 1 + l:2 + l], 0.0)

    acc1 = None
    for q in range(NQ):
        t = jnp.dot(phar_ref[:, q, :], matsn_ref[q * DIM:(q + 1) * DIM, :],
                    preferred_element_type=f32)
        acc1 = t if acc1 is None else acc1 + t
    h1 = _gelu(acc1 + vec_ref[0:1, :])
    prompt = jnp.dot(h1, matsn_ref[NQ * DIM:NQ * DIM + DIM, :],
                     preferred_element_type=f32) + vec_ref[1:2, :]
    mol = _layernorm(prompt + mol_ref[...], vec_ref[2:3, :], vec_ref[3:4, :])
    molt = jnp.transpose(mol)

    wat = wat_ref[...]
    ht = jnp.maximum(jnp.dot(wat, molt, preferred_element_type=f32)
                     + ba_col, 0.0)
    hst = jnp.maximum(jnp.dot(wat, xs, preferred_element_type=f32)
                      + ba_col, 0.0)
    ht_tiled = jnp.concatenate([ht] * SEQ, axis=1)
    ones_d = jnp.ones((1, DIM), f32)
    sig = jnp.dot(ones_d, ht_tiled * hst, preferred_element_type=f32)
    wts = jnp.tanh(sig)
    wprod = wts * hst
    prott = wprod[:, 0:bb]
    for s in range(1, SEQ):
        prott = prott + wprod[:, s * bb:(s + 1) * bb]
    prott = prott * (1.0 / SEQ)

    D2 = 2 * DIM
    cat = jnp.maximum(
        jnp.dot(packh_ref[0:D2, 0:DIM], molt, preferred_element_type=f32)
        + jnp.dot(packh_ref[0:D2, DIM:D2], prott, preferred_element_type=f32)
        + packh_ref[LOUT * D2:LOUT * D2 + D2, 0:1], 0.0)
    for j in range(1, LOUT):
        wjt = packh_ref[j * D2:(j + 1) * D2, :]
        cat = jnp.maximum(
            jnp.dot(wjt, cat, preferred_element_type=f32)
            + packh_ref[LOUT * D2:LOUT * D2 + D2, j:j + 1],
            0.0)

    ones_2d = jnp.ones((1, D2), f32)
    wint_col = packh_ref[LOUT * D2:LOUT * D2 + D2, LOUT:LOUT + 1]
    out = (jnp.dot(ones_2d, cat * wint_col, preferred_element_type=f32)
           + vec_ref[4:5, 0:1])
    out_ref[...] = out


@jax.jit
def _forward(phar_prompt, mol_repr, protein_batch, proj_w1, proj_b1, proj_w2,
             proj_b2, emb, mol_gamma, mol_beta, prot_gamma, prot_beta, conv_w,
             conv_b, wa, ba, wout_w, wout_b, wint_w, wint_b):
    bn = mol_repr.shape[0]
    bb = math.gcd(bn, 2048)
    nblk = bn // bb

    prot_flat = protein_batch.reshape(nblk, bb, SEQ).transpose(0, 2, 1) \
                             .reshape(nblk, 1, SEQ * bb)

    band = _band_cat(conv_w)
    emb_ln = _layernorm(emb, prot_gamma, prot_beta)
    t1 = jnp.dot(emb_ln, band[0])
    rows = []
    for l in range(LCNN):
        base = t1 if l == 0 else band[l]
        rows.append(jnp.concatenate(
            [base[:, di * DIM:(di + 1) * DIM].T for di in range(KTAP)],
            axis=1))
    packc = jnp.concatenate(rows, axis=0)
    wat = wa.T

    matsn = jnp.concatenate([proj_w1, proj_w2],
                            axis=0)
    vec = jnp.concatenate([
        proj_b1, proj_b2, mol_gamma, mol_beta,
        jnp.pad(wint_b, ((0, 0), (0, DIM - 1))),
    ], axis=0)
    vect = jnp.concatenate([
        ba.T,
        jnp.broadcast_to(conv_b[0], (DIM, 1)),
        jnp.broadcast_to(conv_b[1], (DIM, 1)),
        jnp.broadcast_to(conv_b[2], (DIM, 1)),
        jnp.zeros((DIM, 1), jnp.float32),
    ], axis=1)

    D2 = 2 * DIM
    headmats = jnp.concatenate(
        [wout_w[j].T for j in range(LOUT)], axis=0)
    bias_cols = jnp.concatenate(
        [wout_b[j].T for j in range(LOUT)] + [wint_w], axis=1)
    packh = jnp.concatenate([
        headmats,
        jnp.pad(bias_cols, ((0, 0), (0, D2 - (LOUT + 1)))),
    ], axis=0)

    out = pl.pallas_call(
        _dti_block_kernel,
        out_shape=jax.ShapeDtypeStruct((1, bn), jnp.float32),
        grid=(nblk,),
        in_specs=[
            pl.BlockSpec((bb, NQ, DIM), lambda b: (b, 0, 0)),
            pl.BlockSpec((bb, DIM), lambda b: (b, 0)),
            pl.BlockSpec((1, 1, SEQ * bb), lambda b: (b, 0, 0)),
            pl.BlockSpec((LCNN * DIM, KTAP * DIM), lambda b: (0, 0)),
            pl.BlockSpec((DIM, DIM), lambda b: (0, 0)),
            pl.BlockSpec(((NQ + 1) * DIM, DIM), lambda b: (0, 0)),
            pl.BlockSpec((5, DIM), lambda b: (0, 0)),
            pl.BlockSpec((DIM, 5), lambda b: (0, 0)),
            pl.BlockSpec((4 * D2, D2), lambda b: (0, 0)),
        ],
        out_specs=pl.BlockSpec((1, bb), lambda b: (0, b)),
        compiler_params=pltpu.CompilerParams(
            dimension_semantics=("parallel",)),
    )(phar_prompt, mol_repr, prot_flat, packc, wat, matsn, vec, vect, packh)
    return out.reshape(bn, 1)


def kernel(phar_prompt, mol_repr, protein_batch, proj_w1, proj_b1, proj_w2,
           proj_b2, emb, mol_gamma, mol_beta, prot_gamma, prot_beta, conv_w,
           conv_b, wa, ba, wout_w, wout_b, wint_w, wint_b):
    return _forward(phar_prompt, mol_repr, protein_batch, proj_w1, proj_b1,
                    proj_w2, proj_b2, emb, mol_gamma, mol_beta, prot_gamma,
                    prot_beta, conv_w, conv_b, wa, ba, wout_w, wout_b,
                    wint_w, wint_b)


# speedup vs baseline: 2.0402x; 1.3636x over previous
import math

import jax
import jax.numpy as jnp
from jax import lax
from jax.experimental import pallas as pl
from jax.experimental.pallas import tpu as pltpu

SEQ = 16
DIM = 32
NQ = 3
NWORD = 32
WIN = 2
KTAP = 2 * WIN + 1
LCNN = 3
LOUT = 3
LN_EPS = 1e-5


def _layernorm(x, g, b):
    mu = jnp.mean(x, axis=-1, keepdims=True)
    var = jnp.mean((x - mu) ** 2, axis=-1, keepdims=True)
    return (x - mu) * lax.rsqrt(var + LN_EPS) * g + b


def _gelu(x):
    return 0.5 * x * (1.0 + lax.erf(x * 0.7071067811865476))


def _band_cat(conv_w):
    w = conv_w.reshape(LCNN, KTAP, KTAP)
    c = jnp.arange(DIM)[:, None]
    d = jnp.arange(DIM)[None, :]
    dj = c - d + WIN
    valid = (dj >= 0) & (dj < KTAP)
    djc = jnp.clip(dj, 0, KTAP - 1)
    band = jnp.where(valid[None, None], w[:, :, djc], 0.0)
    return band.transpose(0, 2, 1, 3).reshape(LCNN, DIM, KTAP * DIM)


def _shift_lanes(x, sh):
    if sh == 0:
        return x
    nl = x.shape[1]
    z = jnp.zeros((x.shape[0], abs(sh)), x.dtype)
    if sh > 0:
        return jnp.concatenate([x[:, sh:], z], axis=1)
    return jnp.concatenate([z, x[:, :nl + sh]], axis=1)


def _dti_block_kernel(phar_ref, mol_ref, prot_ref, packc_ref, wat_ref,
                      matsn_ref, vec_ref, vect_ref, packh_ref, out_ref):
    bb = phar_ref.shape[0]
    n = SEQ * bb
    f32 = jnp.float32

    idx = prot_ref[0]
    iota = lax.broadcasted_iota(jnp.int32, (NWORD, n), 0)
    xs = (idx == iota).astype(f32)
    ba_col = vect_ref[:, 0:1]
    for l in range(LCNN):
        stack = jnp.concatenate(
            [_shift_lanes(xs, (di - WIN) * bb) for di in range(KTAP)],
            axis=0)
        wc = packc_ref[l * DIM:(l + 1) * DIM, :]
        xs = jnp.maximum(
            jnp.dot(wc, stack, preferred_element_type=f32)
            + vect_ref[:, 1 + l:2 + l], 0.0)

    p = phar_ref[...]
    h1 = _gelu(jnp.dot(p, matsn_ref[0:NQ * DIM, :],
                       preferred_element_type=f32) + vec_ref[0:1, :])
    prompt = jnp.dot(h1, matsn_ref[NQ * DIM:NQ * DIM + DIM, :],
                     preferred_element_type=f32) + vec_ref[1:2, :]
    mol = _layernorm(prompt + mol_ref[...], vec_ref[2:3, :], vec_ref[3:4, :])
    molt = jnp.transpose(mol)

    wat = wat_ref[...]
    ht = jnp.maximum(jnp.dot(wat, molt, preferred_element_type=f32)
                     + ba_col, 0.0)
    hst = jnp.maximum(jnp.dot(wat, xs, preferred_element_type=f32)
                      + ba_col, 0.0)
    ht_tiled = jnp.concatenate([ht] * SEQ, axis=1)
    ones_d = jnp.ones((1, DIM), f32)
    sig = jnp.dot(ones_d, ht_tiled * hst, preferred_element_type=f32)
    wts = jnp.tanh(sig)
    wprod = wts * hst
    prott = wprod[:, 0:bb]
    for s in range(1, SEQ):
        prott = prott + wprod[:, s * bb:(s + 1) * bb]
    prott = prott * (1.0 / SEQ)

    D2 = 2 * DIM
    cat = jnp.maximum(
        jnp.dot(packh_ref[0:D2, 0:DIM], molt, preferred_element_type=f32)
        + jnp.dot(packh_ref[0:D2, DIM:D2], prott, preferred_element_type=f32)
        + packh_ref[LOUT * D2:LOUT * D2 + D2, 0:1], 0.0)
    for j in range(1, LOUT):
        wjt = packh_ref[j * D2:(j + 1) * D2, :]
        cat = jnp.maximum(
            jnp.dot(wjt, cat, preferred_element_type=f32)
            + packh_ref[LOUT * D2:LOUT * D2 + D2, j:j + 1],
            0.0)

    ones_2d = jnp.ones((1, D2), f32)
    wint_col = packh_ref[LOUT * D2:LOUT * D2 + D2, LOUT:LOUT + 1]
    out = (jnp.dot(ones_2d, cat * wint_col, preferred_element_type=f32)
           + vec_ref[4:5, 0:1])
    out_ref[...] = out


@jax.jit
def _forward(phar_prompt, mol_repr, protein_batch, proj_w1, proj_b1, proj_w2,
             proj_b2, emb, mol_gamma, mol_beta, prot_gamma, prot_beta, conv_w,
             conv_b, wa, ba, wout_w, wout_b, wint_w, wint_b):
    bn = mol_repr.shape[0]
    bb = math.gcd(bn, 2048)
    nblk = bn // bb

    phar2 = phar_prompt.reshape(bn, NQ * DIM)
    prot_flat = protein_batch.reshape(nblk, bb, SEQ).transpose(0, 2, 1) \
                             .reshape(nblk, 1, SEQ * bb)

    band = _band_cat(conv_w)
    emb_ln = _layernorm(emb, prot_gamma, prot_beta)
    t1 = jnp.dot(emb_ln, band[0])
    rows = []
    for l in range(LCNN):
        base = t1 if l == 0 else band[l]
        rows.append(jnp.concatenate(
            [base[:, di * DIM:(di + 1) * DIM].T for di in range(KTAP)],
            axis=1))
    packc = jnp.concatenate(rows, axis=0)
    wat = wa.T

    matsn = jnp.concatenate([proj_w1, proj_w2],
                            axis=0)
    vec = jnp.concatenate([
        proj_b1, proj_b2, mol_gamma, mol_beta,
        jnp.pad(wint_b, ((0, 0), (0, DIM - 1))),
    ], axis=0)
    vect = jnp.concatenate([
        ba.T,
        jnp.broadcast_to(conv_b[0], (DIM, 1)),
        jnp.broadcast_to(conv_b[1], (DIM, 1)),
        jnp.broadcast_to(conv_b[2], (DIM, 1)),
        jnp.zeros((DIM, 1), jnp.float32),
    ], axis=1)

    D2 = 2 * DIM
    headmats = jnp.concatenate(
        [wout_w[j].T for j in range(LOUT)], axis=0)
    bias_cols = jnp.concatenate(
        [wout_b[j].T for j in range(LOUT)] + [wint_w], axis=1)
    packh = jnp.concatenate([
        headmats,
        jnp.pad(bias_cols, ((0, 0), (0, D2 - (LOUT + 1)))),
    ], axis=0)

    out = pl.pallas_call(
        _dti_block_kernel,
        out_shape=jax.ShapeDtypeStruct((1, bn), jnp.float32),
        grid=(nblk,),
        in_specs=[
            pl.BlockSpec((bb, NQ * DIM), lambda b: (b, 0)),
            pl.BlockSpec((bb, DIM), lambda b: (b, 0)),
            pl.BlockSpec((1, 1, SEQ * bb), lambda b: (b, 0, 0)),
            pl.BlockSpec((LCNN * DIM, KTAP * DIM), lambda b: (0, 0)),
            pl.BlockSpec((DIM, DIM), lambda b: (0, 0)),
            pl.BlockSpec(((NQ + 1) * DIM, DIM), lambda b: (0, 0)),
            pl.BlockSpec((5, DIM), lambda b: (0, 0)),
            pl.BlockSpec((DIM, 5), lambda b: (0, 0)),
            pl.BlockSpec((4 * D2, D2), lambda b: (0, 0)),
        ],
        out_specs=pl.BlockSpec((1, bb), lambda b: (0, b)),
        compiler_params=pltpu.CompilerParams(
            dimension_semantics=("parallel",)),
    )(phar2, mol_repr, prot_flat, packc, wat, matsn, vec, vect, packh)
    return out.reshape(bn, 1)


def kernel(phar_prompt, mol_repr, protein_batch, proj_w1, proj_b1, proj_w2,
           proj_b2, emb, mol_gamma, mol_beta, prot_gamma, prot_beta, conv_w,
           conv_b, wa, ba, wout_w, wout_b, wint_w, wint_b):
    return _forward(phar_prompt, mol_repr, protein_batch, proj_w1, proj_b1,
                    proj_w2, proj_b2, emb, mol_gamma, mol_beta, prot_gamma,
                    prot_beta, conv_w, conv_b, wa, ba, wout_w, wout_b,
                    wint_w, wint_b)
